# baseline (speedup 1.0000x reference)
import functools

import jax
import jax.numpy as jnp
from jax import lax
from jax.experimental import pallas as pl
from jax.experimental.pallas import tpu as pltpu
from jax.experimental.pallas import tpu_sc as plsc

NP = 10240
BN = 1024
NC = 2
NS = 16
K = 80


def _make_agg(E, FH, with_counts):
    EPT = E // NS
    NCH = EPT // K
    HALF = NCH // 2
    RPT = NP // NS
    mesh = plsc.VectorSubcoreMesh(core_axis_name="c", subcore_axis_name="s")

    out_type = [jax.ShapeDtypeStruct((NC, NP, FH), jnp.float32)]
    if with_counts:
        out_type.append(jax.ShapeDtypeStruct((NC, NP), jnp.float32))

    scratch = [
        pltpu.VMEM((NCH, K), jnp.int32),
        pltpu.VMEM((K,), jnp.int32),
        pltpu.VMEM((K,), jnp.int32),
        pltpu.VMEM((K, FH), jnp.float32),
        pltpu.VMEM((K, FH), jnp.float32),
        pltpu.VMEM((K,), jnp.float32),
        pltpu.VMEM_SHARED((NP, FH), jnp.float32),
        pltpu.VMEM_SHARED((NP, FH), jnp.float32),
        pltpu.VMEM_SHARED((NP,), jnp.float32),
        pltpu.SemaphoreType.DMA,
        pltpu.SemaphoreType.DMA,
        pltpu.SemaphoreType.DMA,
        pltpu.SemaphoreType.DMA,
        pltpu.SemaphoreType.DMA,
        pltpu.SemaphoreType.DMA,
    ]

    def body(p_hbm, src_hbm, dst_hbm, z2_hbm, z1_hbm, ones_hbm,
             *refs):
        if with_counts:
            s_hbm, cnt_hbm = refs[0], refs[1]
            scr = refs[2:]
        else:
            s_hbm = refs[0]
            cnt_hbm = None
            scr = refs[1:]
        (srcidx, dst_a, dst_b, rows_a, rows_b, onesbuf, tbl, acc, cntacc,
         sem_a, sem_b, sem_da, sem_db, sem_sa, sem_sb) = scr
        c = lax.axis_index("c")
        s = lax.axis_index("s")
        row0 = s * RPT
        pltpu.sync_copy(src_hbm.at[2 * s], srcidx.at[pl.ds(0, HALF)])
        pltpu.sync_copy(src_hbm.at[2 * s + 1], srcidx.at[pl.ds(HALF, HALF)])
        pltpu.sync_copy(p_hbm.at[c].at[pl.ds(row0, RPT)],
                        tbl.at[pl.ds(row0, RPT)])
        pltpu.sync_copy(z2_hbm, acc.at[pl.ds(row0, RPT)])
        if with_counts:
            pltpu.sync_copy(z1_hbm, cntacc.at[pl.ds(row0, RPT)])
            pltpu.sync_copy(ones_hbm, onesbuf)
        plsc.subcore_barrier()

        def gather(j, rows, sem):
            pltpu.async_copy(tbl.at[srcidx.at[j]], rows, sem)

        ebase = s * EPT

        def dcopy(j, dbuf, sem):
            pltpu.async_copy(dst_hbm.at[pl.ds(ebase + j * K, K)], dbuf, sem)

        def gwait(rows, dbuf, sem, dsem):
            pltpu.make_async_copy(tbl.at[srcidx.at[0]], rows, sem).wait()
            pltpu.make_async_copy(dst_hbm.at[pl.ds(0, K)], dbuf, dsem).wait()

        def scat(rows, dbuf, sem):
            pltpu.async_copy(rows, acc.at[dbuf], sem, add=True)
            if with_counts:
                pltpu.async_copy(onesbuf, cntacc.at[dbuf], sem, add=True)

        def swait(rows, dbuf, sem):
            pltpu.make_async_copy(rows, acc.at[dbuf], sem).wait()
            if with_counts:
                pltpu.make_async_copy(onesbuf, cntacc.at[dbuf], sem).wait()

        def fetch_b(j, carry):
            gather(j, rows_b, sem_b)
            dcopy(j, dst_b, sem_db)
            return carry

        def fetch_a(j, carry):
            gather(j, rows_a, sem_a)
            dcopy(j, dst_a, sem_da)
            return carry

        fetch_a(0, 0)
        lax.fori_loop(1, 2, fetch_b, 0)

        def step(i, carry):
            j = 2 * i
            gwait(rows_a, dst_a, sem_a, sem_da)
            scat(rows_a, dst_a, sem_sa)
            gwait(rows_b, dst_b, sem_b, sem_db)
            scat(rows_b, dst_b, sem_sb)
            swait(rows_a, dst_a, sem_sa)
            fetch_a(j + 2, 0)
            swait(rows_b, dst_b, sem_sb)
            fetch_b(j + 3, 0)
            return carry

        lax.fori_loop(0, (NCH - 2) // 2, step, 0)
        gwait(rows_a, dst_a, sem_a, sem_da)
        scat(rows_a, dst_a, sem_sa)
        gwait(rows_b, dst_b, sem_b, sem_db)
        scat(rows_b, dst_b, sem_sb)
        swait(rows_a, dst_a, sem_sa)
        swait(rows_b, dst_b, sem_sb)
        plsc.subcore_barrier()
        pltpu.sync_copy(acc.at[pl.ds(row0, RPT)],
                        s_hbm.at[c].at[pl.ds(row0, RPT)])
        if with_counts:
            pltpu.sync_copy(cntacc.at[pl.ds(row0, RPT)],
                            cnt_hbm.at[c].at[pl.ds(row0, RPT)])

    return pl.kernel(body, out_type=out_type, mesh=mesh,
                     scratch_types=scratch,
                     compiler_params=pltpu.CompilerParams(
                         use_tc_tiling_on_sc=False))


def _mm_split_body(h1, x_ref, w_ref, p_ref, r_ref):
    acc = jnp.dot(x_ref[...], w_ref[...],
                  preferred_element_type=jnp.float32,
                  precision=lax.Precision.HIGHEST)
    hh = h1 // 2
    p_ref[0] = acc[:, :hh]
    p_ref[1] = acc[:, hh:h1]
    r_ref[...] = acc[:, h1:]


def _mm_split(x, w, h1):
    n, d = x.shape
    h2 = w.shape[1] - h1
    grid = n // BN
    return pl.pallas_call(
        functools.partial(_mm_split_body, h1),
        grid=(grid,),
        in_specs=[pl.BlockSpec((BN, d), lambda i: (i, 0)),
                  pl.BlockSpec(w.shape, lambda i: (0, 0))],
        out_specs=[pl.BlockSpec((NC, BN, h1 // 2), lambda i: (0, i, 0)),
                   pl.BlockSpec((BN, h2), lambda i: (i, 0))],
        out_shape=[jax.ShapeDtypeStruct((NC, n, h1 // 2), jnp.float32),
                   jax.ShapeDtypeStruct((n, h2), jnp.float32)],
    )(x, w)


def _layer_mid_body(h1, sp_ref, cnt_ref, b_ref, r_ref, w_ref,
                    p_ref, r2_ref):
    ssum = jnp.concatenate([sp_ref[0], sp_ref[1]], axis=1)
    inv = 1.0 / jnp.maximum(cnt_ref[0], 1.0)
    h = jnp.maximum(ssum * inv[:, None] + b_ref[...] + r_ref[...], 0.0)
    acc = jnp.dot(h, w_ref[...], preferred_element_type=jnp.float32,
                  precision=lax.Precision.HIGHEST)
    hh = h1 // 2
    p_ref[0] = acc[:, :hh]
    p_ref[1] = acc[:, hh:h1]
    r2_ref[...] = acc[:, h1:]


def _layer_mid(sp, cnt, b, r, w, h1):
    n = sp.shape[1]
    fh = sp.shape[2]
    h2 = w.shape[1] - h1
    grid = n // BN
    return pl.pallas_call(
        functools.partial(_layer_mid_body, h1),
        grid=(grid,),
        in_specs=[pl.BlockSpec((NC, BN, fh), lambda i: (0, i, 0)),
                  pl.BlockSpec((NC, BN), lambda i: (0, i)),
                  pl.BlockSpec((1, 2 * fh), lambda i: (0, 0)),
                  pl.BlockSpec((BN, 2 * fh), lambda i: (i, 0)),
                  pl.BlockSpec(w.shape, lambda i: (0, 0))],
        out_specs=[pl.BlockSpec((NC, BN, h1 // 2), lambda i: (0, i, 0)),
                   pl.BlockSpec((BN, h2), lambda i: (i, 0))],
        out_shape=[jax.ShapeDtypeStruct((NC, n, h1 // 2), jnp.float32),
                   jax.ShapeDtypeStruct((n, h2), jnp.float32)],
    )(sp, cnt, b, r, w)


def _final_body(sp_ref, cnt_ref, b_ref, r_ref, o_ref):
    ssum = jnp.concatenate([sp_ref[0], sp_ref[1]], axis=1)
    inv = 1.0 / jnp.maximum(cnt_ref[0], 1.0)
    o_ref[...] = ssum * inv[:, None] + b_ref[...] + r_ref[...]


def _final(sp, cnt, b, r):
    n = sp.shape[1]
    fh = sp.shape[2]
    grid = n // BN
    return pl.pallas_call(
        _final_body,
        grid=(grid,),
        in_specs=[pl.BlockSpec((NC, BN, fh), lambda i: (0, i, 0)),
                  pl.BlockSpec((NC, BN), lambda i: (0, i)),
                  pl.BlockSpec((1, 2 * fh), lambda i: (0, 0)),
                  pl.BlockSpec((BN, 2 * fh), lambda i: (i, 0))],
        out_specs=pl.BlockSpec((BN, 2 * fh), lambda i: (i, 0)),
        out_shape=jax.ShapeDtypeStruct((n, 2 * fh), jnp.float32),
    )(sp, cnt, b, r)


def kernel(x, edge_index, W1l, W1r, b1, W2l, W2r, b2):
    n, d = x.shape
    h = W1l.shape[1]
    c = W2l.shape[1]
    e = edge_index.shape[1]

    xp = jnp.pad(x, ((0, NP - n), (0, 0)))
    nw = NC * NS
    nch = e // (nw * K)
    src = edge_index[0].reshape(nw, nch, K)
    dst = edge_index[1]
    z2h = jnp.zeros((NP // NS, h // 2), jnp.float32)
    z2c = jnp.zeros((NP // NS, c // 2), jnp.float32)
    z1 = jnp.zeros((NP // NS,), jnp.float32)
    ones = jnp.ones((K,), jnp.float32)

    w1 = jnp.concatenate([W1l, W1r], axis=1)
    w2 = jnp.concatenate([W2l, W2r], axis=1)

    p1, r1 = _mm_split(xp, w1, h)
    s1, cnt = _make_agg(e, h // 2, True)(p1, src, dst, z2h, z1, ones)
    p2, r2 = _layer_mid(s1, cnt, b1.reshape(1, -1), r1, w2, c)
    (s2,) = _make_agg(e, c // 2, False)(p2, src, dst, z2c, z1, ones)
    out = _final(s2, cnt, b2.reshape(1, -1), r2)
    return out[:n]


# speedup vs baseline: 1.1009x; 1.1009x over previous
import functools

import jax
import jax.numpy as jnp
from jax import lax
from jax.experimental import pallas as pl
from jax.experimental.pallas import tpu as pltpu
from jax.experimental.pallas import tpu_sc as plsc

NP = 10240
BN = 1024
NC = 2
NS = 16
K = 80


def _make_agg(E, F, with_counts, staged=False):
    NW = NC * NS
    EPW = E // NW
    NCH = EPW // K
    RPT = NP // NS
    mesh = plsc.VectorSubcoreMesh(core_axis_name="c", subcore_axis_name="s")

    out_type = [jax.ShapeDtypeStruct((NC, NP, F), jnp.float32)]
    if with_counts:
        out_type.append(jax.ShapeDtypeStruct((NC, NP), jnp.float32))

    scratch = [
        pltpu.VMEM((NCH, K), jnp.int32),
        pltpu.VMEM((K,), jnp.int32),
        pltpu.VMEM((K,), jnp.int32),
        pltpu.VMEM((K, F), jnp.float32),
        pltpu.VMEM((K, F), jnp.float32),
        pltpu.VMEM((K,), jnp.float32),
        pltpu.VMEM_SHARED((NP, F) if staged else (1, 1), jnp.float32),
        pltpu.VMEM_SHARED((NP, F), jnp.float32),
        pltpu.VMEM_SHARED((NP,), jnp.float32),
        pltpu.SemaphoreType.DMA,
        pltpu.SemaphoreType.DMA,
        pltpu.SemaphoreType.DMA,
        pltpu.SemaphoreType.DMA,
        pltpu.SemaphoreType.DMA,
        pltpu.SemaphoreType.DMA,
    ]

    def body(p_hbm, src_hbm, dst_hbm, z2_hbm, z1_hbm, ones_hbm,
             *refs):
        if with_counts:
            s_hbm, cnt_hbm = refs[0], refs[1]
            scr = refs[2:]
        else:
            s_hbm = refs[0]
            cnt_hbm = None
            scr = refs[1:]
        (srcidx, dst_a, dst_b, rows_a, rows_b, onesbuf, tbl, acc, cntacc,
         sem_a, sem_b, sem_da, sem_db, sem_sa, sem_sb) = scr
        c = lax.axis_index("c")
        s = lax.axis_index("s")
        w = s * NC + c
        row0 = s * RPT
        pltpu.sync_copy(src_hbm.at[w], srcidx)
        if staged:
            pltpu.sync_copy(p_hbm.at[pl.ds(row0, RPT)],
                            tbl.at[pl.ds(row0, RPT)])
        pltpu.sync_copy(z2_hbm, acc.at[pl.ds(row0, RPT)])
        if with_counts:
            pltpu.sync_copy(z1_hbm, cntacc.at[pl.ds(row0, RPT)])
            pltpu.sync_copy(ones_hbm, onesbuf)
        plsc.subcore_barrier()

        gsrc = tbl if staged else p_hbm

        def gather(j, rows, sem):
            pltpu.async_copy(gsrc.at[srcidx.at[j]], rows, sem)

        ebase = w * EPW

        def dcopy(j, dbuf, sem):
            pltpu.async_copy(dst_hbm.at[pl.ds(ebase + j * K, K)], dbuf, sem)

        def gwait(rows, dbuf, sem, dsem):
            pltpu.make_async_copy(gsrc.at[srcidx.at[0]], rows, sem).wait()
            pltpu.make_async_copy(dst_hbm.at[pl.ds(0, K)], dbuf, dsem).wait()

        def scat(rows, dbuf, sem):
            pltpu.async_copy(rows, acc.at[dbuf], sem, add=True)
            if with_counts:
                pltpu.async_copy(onesbuf, cntacc.at[dbuf], sem, add=True)

        def swait(rows, dbuf, sem):
            pltpu.make_async_copy(rows, acc.at[dbuf], sem).wait()
            if with_counts:
                pltpu.make_async_copy(onesbuf, cntacc.at[dbuf], sem).wait()

        def fetch_b(j, carry):
            gather(j, rows_b, sem_b)
            dcopy(j, dst_b, sem_db)
            return carry

        def fetch_a(j, carry):
            gather(j, rows_a, sem_a)
            dcopy(j, dst_a, sem_da)
            return carry

        fetch_a(0, 0)
        lax.fori_loop(1, 2, fetch_b, 0)

        def step(i, carry):
            j = 2 * i
            gwait(rows_a, dst_a, sem_a, sem_da)
            scat(rows_a, dst_a, sem_sa)
            gwait(rows_b, dst_b, sem_b, sem_db)
            scat(rows_b, dst_b, sem_sb)
            swait(rows_a, dst_a, sem_sa)
            fetch_a(j + 2, 0)
            swait(rows_b, dst_b, sem_sb)
            fetch_b(j + 3, 0)
            return carry

        lax.fori_loop(0, (NCH - 3) // 2, step, 0)
        gwait(rows_a, dst_a, sem_a, sem_da)
        scat(rows_a, dst_a, sem_sa)
        gwait(rows_b, dst_b, sem_b, sem_db)
        scat(rows_b, dst_b, sem_sb)
        swait(rows_a, dst_a, sem_sa)
        lax.fori_loop(NCH - 1, NCH, fetch_a, 0)
        gwait(rows_a, dst_a, sem_a, sem_da)
        scat(rows_a, dst_a, sem_sa)
        swait(rows_b, dst_b, sem_sb)
        swait(rows_a, dst_a, sem_sa)
        plsc.subcore_barrier()
        pltpu.sync_copy(acc.at[pl.ds(row0, RPT)],
                        s_hbm.at[c].at[pl.ds(row0, RPT)])
        if with_counts:
            pltpu.sync_copy(cntacc.at[pl.ds(row0, RPT)],
                            cnt_hbm.at[c].at[pl.ds(row0, RPT)])

    return pl.kernel(body, out_type=out_type, mesh=mesh,
                     scratch_types=scratch,
                     compiler_params=pltpu.CompilerParams(
                         use_tc_tiling_on_sc=(F % 128 == 0)))


def _mm_split_body(h1, x_ref, w_ref, p_ref, r_ref):
    acc = jnp.dot(x_ref[...], w_ref[...],
                  preferred_element_type=jnp.float32,
                  precision=lax.Precision.HIGHEST)
    p_ref[...] = acc[:, :h1]
    r_ref[...] = acc[:, h1:]


def _mm_split(x, w, h1):
    n, d = x.shape
    h2 = w.shape[1] - h1
    grid = n // BN
    return pl.pallas_call(
        functools.partial(_mm_split_body, h1),
        grid=(grid,),
        in_specs=[pl.BlockSpec((BN, d), lambda i: (i, 0)),
                  pl.BlockSpec(w.shape, lambda i: (0, 0))],
        out_specs=[pl.BlockSpec((BN, h1), lambda i: (i, 0)),
                   pl.BlockSpec((BN, h2), lambda i: (i, 0))],
        out_shape=[jax.ShapeDtypeStruct((n, h1), jnp.float32),
                   jax.ShapeDtypeStruct((n, h2), jnp.float32)],
    )(x, w)


def _layer_mid_body(h1, sp_ref, cnt_ref, b_ref, r_ref, w_ref,
                    p_ref, r2_ref):
    ssum = sp_ref[0] + sp_ref[1]
    cnt = cnt_ref[0] + cnt_ref[1]
    inv = 1.0 / jnp.maximum(cnt, 1.0)
    h = jnp.maximum(ssum * inv[:, None] + b_ref[...] + r_ref[...], 0.0)
    acc = jnp.dot(h, w_ref[...], preferred_element_type=jnp.float32,
                  precision=lax.Precision.HIGHEST)
    p_ref[...] = acc[:, :h1]
    r2_ref[...] = acc[:, h1:]


def _layer_mid(sp, cnt, b, r, w, h1):
    n = sp.shape[1]
    d = sp.shape[2]
    h2 = w.shape[1] - h1
    grid = n // BN
    return pl.pallas_call(
        functools.partial(_layer_mid_body, h1),
        grid=(grid,),
        in_specs=[pl.BlockSpec((NC, BN, d), lambda i: (0, i, 0)),
                  pl.BlockSpec((NC, BN), lambda i: (0, i)),
                  pl.BlockSpec((1, d), lambda i: (0, 0)),
                  pl.BlockSpec((BN, d), lambda i: (i, 0)),
                  pl.BlockSpec(w.shape, lambda i: (0, 0))],
        out_specs=[pl.BlockSpec((BN, h1), lambda i: (i, 0)),
                   pl.BlockSpec((BN, h2), lambda i: (i, 0))],
        out_shape=[jax.ShapeDtypeStruct((n, h1), jnp.float32),
                   jax.ShapeDtypeStruct((n, h2), jnp.float32)],
    )(sp, cnt, b, r, w)


def _final_body(sp_ref, cnt_ref, b_ref, r_ref, o_ref):
    ssum = sp_ref[0] + sp_ref[1]
    cnt = cnt_ref[0] + cnt_ref[1]
    inv = 1.0 / jnp.maximum(cnt, 1.0)
    o_ref[...] = ssum * inv[:, None] + b_ref[...] + r_ref[...]


def _final(sp, cnt, b, r):
    n = sp.shape[1]
    c = sp.shape[2]
    grid = n // BN
    return pl.pallas_call(
        _final_body,
        grid=(grid,),
        in_specs=[pl.BlockSpec((NC, BN, c), lambda i: (0, i, 0)),
                  pl.BlockSpec((NC, BN), lambda i: (0, i)),
                  pl.BlockSpec((1, c), lambda i: (0, 0)),
                  pl.BlockSpec((BN, c), lambda i: (i, 0))],
        out_specs=pl.BlockSpec((BN, c), lambda i: (i, 0)),
        out_shape=jax.ShapeDtypeStruct((n, c), jnp.float32),
    )(sp, cnt, b, r)


def kernel(x, edge_index, W1l, W1r, b1, W2l, W2r, b2):
    n, d = x.shape
    h = W1l.shape[1]
    c = W2l.shape[1]
    e = edge_index.shape[1]

    xp = jnp.pad(x, ((0, NP - n), (0, 0)))
    nw = NC * NS
    nch = e // (nw * K)
    src = edge_index[0].reshape(nw, nch, K)
    dst = edge_index[1]
    z2h = jnp.zeros((NP // NS, h), jnp.float32)
    z2c = jnp.zeros((NP // NS, c), jnp.float32)
    z1 = jnp.zeros((NP // NS,), jnp.float32)
    ones = jnp.ones((K,), jnp.float32)

    w1 = jnp.concatenate([W1l, W1r], axis=1)
    w2 = jnp.concatenate([W2l, W2r], axis=1)

    p1, r1 = _mm_split(xp, w1, h)
    s1, cnt = _make_agg(e, h, True)(p1, src, dst, z2h, z1, ones)
    p2, r2 = _layer_mid(s1, cnt, b1.reshape(1, -1), r1, w2, c)
    (s2,) = _make_agg(e, c, False, staged=True)(p2, src, dst, z2c, z1, ones)
    out = _final(s2, cnt, b2.reshape(1, -1), r2)
    return out[:n]
